# baseline (speedup 1.0000x reference)
import jax
import jax.numpy as jnp
from jax import lax
from jax.experimental import pallas as pl
from jax.experimental.pallas import tpu as pltpu
from jax.experimental.pallas import tpu_sc as plsc

B, C, H, W = 32, 192, 64, 64
NSEL = 32
PLANES = B * C
NW = 32
PPW = PLANES // NW
PB = 4
NBATCH = PPW // PB
ROWS = PB * NSEL
L = 16


def _sc_kernel(x_hbm, ri_hbm, rj_hbm, out_hbm, ri_v, rj_v, idx_v, rows_v,
               out_v, gsem, osem):
    nc = 2
    wid = lax.axis_index("s") * nc + lax.axis_index("c")
    base_plane = wid * PPW

    pltpu.sync_copy(ri_hbm, ri_v)
    pltpu.sync_copy(rj_hbm, rj_v)
    ri_lo = ri_v[pl.ds(0, L)]
    ri_hi = ri_v[pl.ds(L, L)]
    rj_lo = rj_v[pl.ds(0, L)]
    rj_hi = rj_v[pl.ds(L, L)]

    def body(t, carry):
        plane0 = base_plane + t * PB
        for b in range(PB):
            rb = (plane0 + b) * W
            idx_v[pl.ds(b * NSEL, L)] = rj_lo + rb
            idx_v[pl.ds(b * NSEL + L, L)] = rj_hi + rb
        pltpu.async_copy(x_hbm.at[idx_v], rows_v, gsem).wait()

        for r in range(ROWS):
            row = rows_v.at[r]
            lo = plsc.load_gather(row, [ri_lo])
            hi = plsc.load_gather(row, [ri_hi])
            out_v[pl.ds(r * NSEL, L)] = lo
            out_v[pl.ds(r * NSEL + L, L)] = hi

        pltpu.async_copy(out_v, out_hbm.at[pl.ds(plane0 * NSEL * NSEL,
                                                 ROWS * NSEL)], osem).wait()
        return carry

    lax.fori_loop(0, NBATCH, body, 0)


def kernel(x, randi, randj):
    x_rows = x.reshape(PLANES * H, W)
    mesh = plsc.VectorSubcoreMesh(core_axis_name="c", subcore_axis_name="s")
    run = pl.kernel(
        _sc_kernel,
        out_type=jax.ShapeDtypeStruct((PLANES * NSEL * NSEL,), jnp.float32),
        mesh=mesh,
        compiler_params=pltpu.CompilerParams(needs_layout_passes=False,
                                             use_tc_tiling_on_sc=False),
        scratch_types=[
            pltpu.VMEM((NSEL,), jnp.int32),
            pltpu.VMEM((NSEL,), jnp.int32),
            pltpu.VMEM((ROWS,), jnp.int32),
            pltpu.VMEM((ROWS, W), jnp.float32),
            pltpu.VMEM((ROWS * NSEL,), jnp.float32),
            pltpu.SemaphoreType.DMA,
            pltpu.SemaphoreType.DMA,
        ],
    )
    out = run(x_rows, randi, randj)
    return out.reshape(B, C, NSEL, NSEL)


# speedup vs baseline: 8.8112x; 8.8112x over previous
import jax
import jax.numpy as jnp
from jax import lax
from jax.experimental import pallas as pl
from jax.experimental.pallas import tpu as pltpu
from jax.experimental.pallas import tpu_sc as plsc

B, C, H, W = 32, 192, 64, 64
NSEL = 32
NW = 32
L = 16
NB = 4


def _sc_kernel(xp_hbm, ri_hbm, rj_hbm, out_hbm, idx_v, slab_v, gsem, osem):
    nc = 2
    wid = lax.axis_index("s") * nc + lax.axis_index("c")
    b = wid

    pltpu.sync_copy(ri_hbm, idx_v.at[0])
    pltpu.sync_copy(rj_hbm, idx_v.at[1])
    ri_lo = idx_v[0, pl.ds(0, L)]
    ri_hi = idx_v[0, pl.ds(L, L)]
    rj_lo = idx_v[1, pl.ds(0, L)]
    rj_hi = idx_v[1, pl.ds(L, L)]
    ws = [ri_lo[j] for j in range(L)] + [ri_hi[j] for j in range(L)]
    lanes = lax.iota(jnp.int32, L)

    def randj_at(i):
        return (jnp.sum(jnp.where(lanes == i, rj_lo, 0))
                + jnp.sum(jnp.where(lanes == i - L, rj_hi, 0)))

    def fire_gather(i, k):
        h = randj_at(i)
        for j in range(NSEL):
            pltpu.async_copy(xp_hbm.at[b, h, ws[j]], slab_v.at[k, j],
                             gsem.at[k])

    def drain_gather(k):
        pltpu.make_async_copy(xp_hbm.at[b, 0, pl.ds(0, NSEL)],
                              slab_v.at[k], gsem.at[k]).wait()

    def drain_out(i, k):
        pltpu.make_async_copy(slab_v.at[k], out_hbm.at[b, i],
                              osem.at[k]).wait()

    fire_gather(0, 0)
    fire_gather(1, 1)

    def step(t, carry):
        for kk in range(NB):
            g = t * NB + kk
            drain_gather(kk)
            pltpu.async_copy(slab_v.at[kk], out_hbm.at[b, g], osem.at[kk])

            @pl.when(g >= 2)
            def _():
                drain_out(g - 2, (kk + 2) % NB)

            @pl.when(g + 2 < NSEL)
            def _():
                fire_gather(g + 2, (kk + 2) % NB)
        return carry

    lax.fori_loop(0, NSEL // NB, step, 0)
    drain_out(NSEL - 2, (NSEL - 2) % NB)
    drain_out(NSEL - 1, (NSEL - 1) % NB)


def kernel(x, randi, randj):
    x_p = jnp.transpose(x, (0, 2, 3, 1))
    mesh = plsc.VectorSubcoreMesh(core_axis_name="c", subcore_axis_name="s")
    run = pl.kernel(
        _sc_kernel,
        out_type=jax.ShapeDtypeStruct((B, NSEL, NSEL, C), jnp.float32),
        mesh=mesh,
        compiler_params=pltpu.CompilerParams(needs_layout_passes=False),
        scratch_types=[
            pltpu.VMEM((2, NSEL), jnp.int32),
            pltpu.VMEM((NB, NSEL, C), jnp.float32),
            pltpu.SemaphoreType.DMA((NB,)),
            pltpu.SemaphoreType.DMA((NB,)),
        ],
    )
    out_p = run(x_p, randi, randj)
    return jnp.transpose(out_p, (0, 3, 1, 2))


# speedup vs baseline: 9.0207x; 1.0238x over previous
import jax
import jax.numpy as jnp
from jax import lax
from jax.experimental import pallas as pl
from jax.experimental.pallas import tpu as pltpu
from jax.experimental.pallas import tpu_sc as plsc

B, C, H, W = 32, 192, 64, 64
NSEL = 32
NW = 32
L = 16
NB = 8
LD = 4


def _sc_kernel(xp_hbm, ri_hbm, rj_hbm, out_hbm, idx_v, slab_v, gsem, osem):
    nc = 2
    wid = lax.axis_index("s") * nc + lax.axis_index("c")
    b = wid

    pltpu.sync_copy(ri_hbm, idx_v.at[0])
    pltpu.sync_copy(rj_hbm, idx_v.at[1])
    ri_lo = idx_v[0, pl.ds(0, L)]
    ri_hi = idx_v[0, pl.ds(L, L)]
    rj_lo = idx_v[1, pl.ds(0, L)]
    rj_hi = idx_v[1, pl.ds(L, L)]
    ws = [ri_lo[j] for j in range(L)] + [ri_hi[j] for j in range(L)]
    lanes = lax.iota(jnp.int32, L)

    def randj_at(i):
        return (jnp.sum(jnp.where(lanes == i, rj_lo, 0))
                + jnp.sum(jnp.where(lanes == i - L, rj_hi, 0)))

    def fire_gather(i, k):
        h = randj_at(i)
        for j in range(NSEL):
            pltpu.async_copy(xp_hbm.at[b, h, ws[j]], slab_v.at[k, j],
                             gsem.at[k])

    def drain_gather(k):
        pltpu.make_async_copy(xp_hbm.at[b, 0, pl.ds(0, NSEL)],
                              slab_v.at[k], gsem.at[k]).wait()

    def drain_out(i, k):
        pltpu.make_async_copy(slab_v.at[k], out_hbm.at[b, i],
                              osem.at[k]).wait()

    for q in range(LD):
        fire_gather(q, q)

    def step(t, carry):
        for kk in range(NB):
            g = t * NB + kk
            drain_gather(kk)
            pltpu.async_copy(slab_v.at[kk], out_hbm.at[b, g], osem.at[kk])

            @pl.when(g >= LD)
            def _():
                drain_out(g - LD, (kk + LD) % NB)

            @pl.when(g + LD < NSEL)
            def _():
                fire_gather(g + LD, (kk + LD) % NB)
        return carry

    lax.fori_loop(0, NSEL // NB, step, 0)
    for q in range(NSEL - LD, NSEL):
        drain_out(q, q % NB)


def kernel(x, randi, randj):
    x_p = jnp.transpose(x, (0, 2, 3, 1))
    mesh = plsc.VectorSubcoreMesh(core_axis_name="c", subcore_axis_name="s")
    run = pl.kernel(
        _sc_kernel,
        out_type=jax.ShapeDtypeStruct((B, NSEL, NSEL, C), jnp.float32),
        mesh=mesh,
        compiler_params=pltpu.CompilerParams(needs_layout_passes=False),
        scratch_types=[
            pltpu.VMEM((2, NSEL), jnp.int32),
            pltpu.VMEM((NB, NSEL, C), jnp.float32),
            pltpu.SemaphoreType.DMA((NB,)),
            pltpu.SemaphoreType.DMA((NB,)),
        ],
    )
    out_p = run(x_p, randi, randj)
    return jnp.transpose(out_p, (0, 3, 1, 2))


# speedup vs baseline: 9.7655x; 1.0826x over previous
import jax
import jax.numpy as jnp
from jax import lax
from jax.experimental import pallas as pl
from jax.experimental.pallas import tpu as pltpu
from jax.experimental.pallas import tpu_sc as plsc

B, C, H, W = 32, 192, 64, 64
NSEL = 32
NW = 32
L = 16
NB = 8
LD = 4


def _sc_kernel(xp_hbm, ri_hbm, rj_hbm, out_hbm, idx_v, slab_v, gsem, osem):
    nc = 2
    wid = lax.axis_index("s") * nc + lax.axis_index("c")
    b = wid

    pltpu.sync_copy(ri_hbm, idx_v.at[0])
    pltpu.sync_copy(rj_hbm, idx_v.at[1])
    ri_lo = idx_v[0, pl.ds(0, L)]
    ri_hi = idx_v[0, pl.ds(L, L)]
    rj_lo = idx_v[1, pl.ds(0, L)]
    rj_hi = idx_v[1, pl.ds(L, L)]
    lanes = lax.iota(jnp.int32, L)

    def vec_at(lo, hi, i):
        return (jnp.sum(jnp.where(lanes == i, lo, 0))
                + jnp.sum(jnp.where(lanes == i - L, hi, 0)))

    def fire_gather(i, k):
        h = vec_at(rj_lo, rj_hi, i)

        def jbody(j, carry):
            w = vec_at(ri_lo, ri_hi, j)
            pltpu.async_copy(xp_hbm.at[b, h, w], slab_v.at[k, j],
                             gsem.at[k])
            return carry

        lax.fori_loop(0, NSEL, jbody, 0)

    def drain_gather(k):
        pltpu.make_async_copy(xp_hbm.at[b, 0, pl.ds(0, NSEL)],
                              slab_v.at[k], gsem.at[k]).wait()

    def drain_out(i, k):
        pltpu.make_async_copy(slab_v.at[k], out_hbm.at[b, i],
                              osem.at[k]).wait()

    for q in range(LD):
        fire_gather(q, q)

    def step(t, carry):
        for kk in range(NB):
            g = t * NB + kk
            drain_gather(kk)
            pltpu.async_copy(slab_v.at[kk], out_hbm.at[b, g], osem.at[kk])

            @pl.when(g >= LD)
            def _():
                drain_out(g - LD, (kk + LD) % NB)

            @pl.when(g + LD < NSEL)
            def _():
                fire_gather(g + LD, (kk + LD) % NB)
        return carry

    lax.fori_loop(0, NSEL // NB, step, 0)
    for q in range(NSEL - LD, NSEL):
        drain_out(q, q % NB)


def kernel(x, randi, randj):
    x_p = jnp.transpose(x, (0, 2, 3, 1))
    mesh = plsc.VectorSubcoreMesh(core_axis_name="c", subcore_axis_name="s")
    run = pl.kernel(
        _sc_kernel,
        out_type=jax.ShapeDtypeStruct((B, NSEL, NSEL, C), jnp.float32),
        mesh=mesh,
        compiler_params=pltpu.CompilerParams(needs_layout_passes=False),
        scratch_types=[
            pltpu.VMEM((2, NSEL), jnp.int32),
            pltpu.VMEM((NB, NSEL, C), jnp.float32),
            pltpu.SemaphoreType.DMA((NB,)),
            pltpu.SemaphoreType.DMA((NB,)),
        ],
    )
    out_p = run(x_p, randi, randj)
    return jnp.transpose(out_p, (0, 3, 1, 2))
